# baseline (speedup 1.0000x reference)
import functools

import jax
import jax.numpy as jnp
from jax.experimental import pallas as pl
from jax.experimental.pallas import tpu as pltpu

_EPS = 1e-5


def _conv_stage_kernel(x_ref, w_ref, b_ref, sc_ref, t_ref, s_ref, o_ref,
                       rhs_ref, y_ref, m_ref, *, k, Wp, L, Cin, S, Cout, n,
                       pad_out):
    def rl(v, s):
        return pltpu.roll(v, L - s, 1)

    x = x_ref[...]
    left = pltpu.roll(x, 1, 1)
    right = rl(x, 1)
    rhs_ref[pl.ds(0, Cin), :] = left
    rhs_ref[pl.ds(S, Cin), :] = x
    rhs_ref[pl.ds(2 * S, Cin), :] = right
    rhs = rhs_ref[...]
    p0 = jnp.dot(w_ref[0], rhs, preferred_element_type=jnp.float32)
    p1 = jnp.dot(w_ref[1], rhs, preferred_element_type=jnp.float32)
    p2 = jnp.dot(w_ref[2], rhs, preferred_element_type=jnp.float32)
    y = pltpu.roll(p0, Wp, 1) + p1 + rl(p2, Wp)
    if k == 2:
        m = jnp.maximum(y, rl(y, 1))
        m = jnp.maximum(m, rl(m, Wp))
    elif k == 3:
        m = jnp.maximum(y, rl(y, 1))
        m = jnp.maximum(m, rl(y, 2))
        m = jnp.maximum(m, jnp.maximum(rl(m, Wp), rl(m, 2 * Wp)))
    else:
        mc = jnp.maximum(y, rl(y, 1))
        mc = jnp.maximum(mc, rl(mc, 2))
        mc = jnp.maximum(mc, rl(y, 4))
        m = jnp.maximum(mc, rl(mc, Wp))
        m = jnp.maximum(m, rl(m, 2 * Wp))
        m = jnp.maximum(m, rl(mc, 4 * Wp))
    hi = m.astype(jnp.bfloat16)
    lo = (m - hi.astype(jnp.float32)).astype(jnp.bfloat16)
    m_ref[pl.ds(0, Cout), :] = hi
    m_ref[pl.ds(Cout, Cout), :] = lo
    s = s_ref[...]
    col = jax.lax.broadcasted_iota(jnp.int32, (Cout, 128), 1)
    live = jnp.logical_and(col >= pad_out, col < pad_out + n)
    if pad_out:
        o_ref[:, pl.ds(0, 128)] = jnp.zeros((Cout, 128), o_ref.dtype)
        o_ref[:, pl.ds((n + 1) * 128, 128)] = jnp.zeros((Cout, 128),
                                                        o_ref.dtype)
    for p in range(n):
        rhi = m_ref[pl.ds(0, Cout), pl.ds((1 + k * p) * Wp, Wp)]
        rlo = m_ref[pl.ds(Cout, Cout), pl.ds((1 + k * p) * Wp, Wp)]
        z = (jnp.dot(rhi, s, preferred_element_type=jnp.float32)
             + jnp.dot(rlo, s, preferred_element_type=jnp.float32))
        z = (jnp.maximum(z + b_ref[...], 0.0) * sc_ref[...] + t_ref[...])
        z = jnp.where(live, z, 0.0)
        o_ref[:, pl.ds((p + pad_out) * 128, 128)] = z.astype(o_ref.dtype)


def _conv_stage(x_pad, w_hwio, b, gamma, beta, rm, rv, *, k, Wp, n,
                pad_out, out_dtype=jnp.bfloat16):
    B, Cin, L = x_pad.shape
    Cout = w_hwio.shape[-1]
    K3 = 3 * Cin

    scale = gamma / jnp.sqrt(rv + _EPS)
    shift = beta - rm * scale
    S = -(-Cin // 16) * 16
    Cw = w_hwio.shape[2]
    wk = jnp.zeros((3, Cout, 3, S), jnp.float32)
    wk = wk.at[:, :, :, :Cw].set(jnp.transpose(w_hwio, (0, 3, 1, 2)))
    wk = wk.reshape(3, Cout, 3 * S).astype(jnp.bfloat16)
    sel = jnp.zeros((Wp, 128), jnp.bfloat16)
    sel = sel.at[1 + k * jnp.arange(n), pad_out + jnp.arange(n)].set(1.0)

    Lo = (n + 2 * pad_out) * 128
    out = pl.pallas_call(
        functools.partial(_conv_stage_kernel, k=k, Wp=Wp, L=L, Cin=Cin,
                          S=S, Cout=Cout, n=n, pad_out=pad_out),
        out_shape=jax.ShapeDtypeStruct((B, Cout, Lo), out_dtype),
        grid_spec=pltpu.PrefetchScalarGridSpec(
            num_scalar_prefetch=0,
            grid=(B,),
            in_specs=[
                pl.BlockSpec((None, Cin, L), lambda i: (i, 0, 0)),
                pl.BlockSpec((3, Cout, 3 * S), lambda i: (0, 0, 0)),
                pl.BlockSpec((Cout, 1), lambda i: (0, 0)),
                pl.BlockSpec((Cout, 1), lambda i: (0, 0)),
                pl.BlockSpec((Cout, 1), lambda i: (0, 0)),
                pl.BlockSpec((Wp, 128), lambda i: (0, 0)),
            ],
            out_specs=pl.BlockSpec((None, Cout, Lo), lambda i: (i, 0, 0)),
            scratch_shapes=[
                pltpu.VMEM((3 * S, L), jnp.bfloat16),
                pltpu.VMEM((Cout, L), jnp.float32),
                pltpu.VMEM((2 * Cout, L), jnp.bfloat16),
            ],
        ),
        compiler_params=pltpu.CompilerParams(
            dimension_semantics=("parallel",)),
    )(x_pad, wk, b.reshape(Cout, 1), scale.reshape(Cout, 1),
      shift.reshape(Cout, 1), sel)
    return out


def _head_kernel(x_ref, w1_ref, b1_ref, s1_ref, t1_ref, w2_ref, b2_ref,
                 o_ref):
    h = jnp.dot(x_ref[...], w1_ref[...], preferred_element_type=jnp.float32)
    h = jnp.maximum(h + b1_ref[...], 0.0)
    h = h * s1_ref[...] + t1_ref[...]
    o_ref[...] = (jnp.dot(h, w2_ref[...], preferred_element_type=jnp.float32)
                  + b2_ref[...])


def _fc_head(x, w1, b1, gamma, beta, rm, rv, w2, b2):
    B, F = x.shape
    N1, N2 = w1.shape[1], w2.shape[1]
    scale = gamma / jnp.sqrt(rv + _EPS)
    shift = beta - rm * scale
    bh = B // 2
    return pl.pallas_call(
        _head_kernel,
        out_shape=jax.ShapeDtypeStruct((B, N2), jnp.float32),
        grid_spec=pltpu.PrefetchScalarGridSpec(
            num_scalar_prefetch=0,
            grid=(2,),
            in_specs=[
                pl.BlockSpec((bh, F), lambda i: (i, 0)),
                pl.BlockSpec((F, N1), lambda i: (0, 0)),
                pl.BlockSpec((1, N1), lambda i: (0, 0)),
                pl.BlockSpec((1, N1), lambda i: (0, 0)),
                pl.BlockSpec((1, N1), lambda i: (0, 0)),
                pl.BlockSpec((N1, N2), lambda i: (0, 0)),
                pl.BlockSpec((1, N2), lambda i: (0, 0)),
            ],
            out_specs=pl.BlockSpec((bh, N2), lambda i: (i, 0)),
        ),
        compiler_params=pltpu.CompilerParams(
            dimension_semantics=("parallel",)),
    )(x, w1, b1.reshape(1, N1), scale.reshape(1, N1), shift.reshape(1, N1),
      w2, b2.reshape(1, N2))


def kernel(x, w1, b1, bn1_gamma, bn1_beta, bn1_rm, bn1_rv,
           w2, b2, bn2_gamma, bn2_beta, bn2_rm, bn2_rv,
           w3, b3, bn3_gamma, bn3_beta, bn3_rm, bn3_rv,
           fc1_w, fc1_b, bn4_gamma, bn4_beta, bn4_rm, bn4_rv,
           fc2_w, fc2_b):
    B, _, H1, W1 = x.shape
    H2, H3, H4 = H1 // 2, H1 // 6, H1 // 30
    Wp1 = -(-(W1 + 2) // 128) * 128

    xi = jnp.pad(x.astype(jnp.bfloat16),
                 ((0, 0), (0, 13), (1, 1), (1, Wp1 - W1 - 1)))
    xi = xi.reshape(B, 16, (H1 + 2) * Wp1)

    xi = _conv_stage(xi, w1, b1, bn1_gamma, bn1_beta, bn1_rm, bn1_rv,
                     k=2, Wp=Wp1, n=H2, pad_out=1)
    xi = _conv_stage(xi, w2, b2, bn2_gamma, bn2_beta, bn2_rm, bn2_rv,
                     k=3, Wp=128, n=H3, pad_out=1)
    o = _conv_stage(xi, w3, b3, bn3_gamma, bn3_beta, bn3_rm, bn3_rv,
                    k=5, Wp=128, n=H4, pad_out=0, out_dtype=jnp.float32)
    o = o.reshape(B, 32, H4, 128)[:, :, :, :H4]
    flat = o.reshape(B, -1)
    return _fc_head(flat, fc1_w, fc1_b, bn4_gamma, bn4_beta, bn4_rm, bn4_rv,
                    fc2_w, fc2_b)


# speedup vs baseline: 1.2199x; 1.2199x over previous
import functools

import jax
import jax.numpy as jnp
from jax.experimental import pallas as pl
from jax.experimental.pallas import tpu as pltpu

_EPS = 1e-5


def _conv_stage_kernel(x_ref, w_ref, b_ref, sc_ref, t_ref, s_ref, o_ref,
                       rhs_ref, y_ref, m_ref, *, k, Wp, L, Cin, Cout, n,
                       pad_out):
    def rl(v, s):
        return pltpu.roll(v, L - s, 1)

    x = x_ref[...]
    left = pltpu.roll(x, 1, 1)
    right = rl(x, 1)
    rhs_ref[pl.ds(0, Cin), :] = left
    rhs_ref[pl.ds(Cin, Cin), :] = x
    rhs_ref[pl.ds(2 * Cin, Cin), :] = right
    rhs = rhs_ref[...]
    p0 = jnp.dot(w_ref[0], rhs, preferred_element_type=jnp.float32)
    p1 = jnp.dot(w_ref[1], rhs, preferred_element_type=jnp.float32)
    p2 = jnp.dot(w_ref[2], rhs, preferred_element_type=jnp.float32)
    y_ref[...] = pltpu.roll(p0, Wp, 1) + p1 + rl(p2, Wp)
    y = y_ref[...]
    if k == 2:
        m = jnp.maximum(y, rl(y, 1))
        m = jnp.maximum(m, rl(m, Wp))
    elif k == 3:
        m = jnp.maximum(y, rl(y, 1))
        m = jnp.maximum(m, rl(y, 2))
        m = jnp.maximum(m, jnp.maximum(rl(m, Wp), rl(m, 2 * Wp)))
    else:
        mc = jnp.maximum(y, rl(y, 1))
        mc = jnp.maximum(mc, rl(mc, 2))
        mc = jnp.maximum(mc, rl(y, 4))
        m = jnp.maximum(mc, rl(mc, Wp))
        m = jnp.maximum(m, rl(m, 2 * Wp))
        m = jnp.maximum(m, rl(mc, 4 * Wp))
    hi = m.astype(jnp.bfloat16)
    lo = (m - hi.astype(jnp.float32)).astype(jnp.bfloat16)
    m_ref[pl.ds(0, Cout), :] = hi
    m_ref[pl.ds(Cout, Cout), :] = lo
    s = s_ref[...]
    col = jax.lax.broadcasted_iota(jnp.int32, (Cout, 128), 1)
    live = jnp.logical_and(col >= pad_out, col < pad_out + n)
    if pad_out:
        o_ref[:, pl.ds(0, 128)] = jnp.zeros((Cout, 128), o_ref.dtype)
        o_ref[:, pl.ds((n + 1) * 128, 128)] = jnp.zeros((Cout, 128),
                                                        o_ref.dtype)
    for p in range(n):
        rhi = m_ref[pl.ds(0, Cout), pl.ds((1 + k * p) * Wp, Wp)]
        rlo = m_ref[pl.ds(Cout, Cout), pl.ds((1 + k * p) * Wp, Wp)]
        z = (jnp.dot(rhi, s, preferred_element_type=jnp.float32)
             + jnp.dot(rlo, s, preferred_element_type=jnp.float32))
        z = (jnp.maximum(z + b_ref[...], 0.0) * sc_ref[...] + t_ref[...])
        z = jnp.where(live, z, 0.0)
        o_ref[:, pl.ds((p + pad_out) * 128, 128)] = z.astype(o_ref.dtype)


def _conv_stage(x_pad, w_hwio, b, gamma, beta, rm, rv, *, k, Wp, n,
                pad_out, out_dtype=jnp.bfloat16):
    B, Cin, L = x_pad.shape
    Cout = w_hwio.shape[-1]
    K3 = 3 * Cin

    scale = gamma / jnp.sqrt(rv + _EPS)
    shift = beta - rm * scale
    wk = jnp.transpose(w_hwio, (0, 3, 1, 2)).reshape(3, Cout, K3)
    wk = wk.astype(jnp.bfloat16)
    sel = jnp.zeros((Wp, 128), jnp.bfloat16)
    sel = sel.at[1 + k * jnp.arange(n), pad_out + jnp.arange(n)].set(1.0)

    Lo = (n + 2 * pad_out) * 128
    out = pl.pallas_call(
        functools.partial(_conv_stage_kernel, k=k, Wp=Wp, L=L, Cin=Cin,
                          Cout=Cout, n=n, pad_out=pad_out),
        out_shape=jax.ShapeDtypeStruct((B, Cout, Lo), out_dtype),
        grid_spec=pltpu.PrefetchScalarGridSpec(
            num_scalar_prefetch=0,
            grid=(B,),
            in_specs=[
                pl.BlockSpec((None, Cin, L), lambda i: (i, 0, 0)),
                pl.BlockSpec((3, Cout, K3), lambda i: (0, 0, 0)),
                pl.BlockSpec((Cout, 1), lambda i: (0, 0)),
                pl.BlockSpec((Cout, 1), lambda i: (0, 0)),
                pl.BlockSpec((Cout, 1), lambda i: (0, 0)),
                pl.BlockSpec((Wp, 128), lambda i: (0, 0)),
            ],
            out_specs=pl.BlockSpec((None, Cout, Lo), lambda i: (i, 0, 0)),
            scratch_shapes=[
                pltpu.VMEM((K3, L), jnp.bfloat16),
                pltpu.VMEM((Cout, L), jnp.float32),
                pltpu.VMEM((2 * Cout, L), jnp.bfloat16),
            ],
        ),
        compiler_params=pltpu.CompilerParams(
            dimension_semantics=("parallel",)),
    )(x_pad, wk, b.reshape(Cout, 1), scale.reshape(Cout, 1),
      shift.reshape(Cout, 1), sel)
    return out


def _head_kernel(x_ref, w1_ref, b1_ref, s1_ref, t1_ref, w2_ref, b2_ref,
                 o_ref):
    h = jnp.dot(x_ref[...], w1_ref[...], preferred_element_type=jnp.float32)
    h = jnp.maximum(h + b1_ref[...], 0.0)
    h = h * s1_ref[...] + t1_ref[...]
    o_ref[...] = (jnp.dot(h, w2_ref[...], preferred_element_type=jnp.float32)
                  + b2_ref[...])


def _fc_head(x, w1, b1, gamma, beta, rm, rv, w2, b2):
    B, F = x.shape
    N1, N2 = w1.shape[1], w2.shape[1]
    scale = gamma / jnp.sqrt(rv + _EPS)
    shift = beta - rm * scale
    bh = B // 2
    return pl.pallas_call(
        _head_kernel,
        out_shape=jax.ShapeDtypeStruct((B, N2), jnp.float32),
        grid_spec=pltpu.PrefetchScalarGridSpec(
            num_scalar_prefetch=0,
            grid=(2,),
            in_specs=[
                pl.BlockSpec((bh, F), lambda i: (i, 0)),
                pl.BlockSpec((F, N1), lambda i: (0, 0)),
                pl.BlockSpec((1, N1), lambda i: (0, 0)),
                pl.BlockSpec((1, N1), lambda i: (0, 0)),
                pl.BlockSpec((1, N1), lambda i: (0, 0)),
                pl.BlockSpec((N1, N2), lambda i: (0, 0)),
                pl.BlockSpec((1, N2), lambda i: (0, 0)),
            ],
            out_specs=pl.BlockSpec((bh, N2), lambda i: (i, 0)),
        ),
        compiler_params=pltpu.CompilerParams(
            dimension_semantics=("parallel",)),
    )(x, w1, b1.reshape(1, N1), scale.reshape(1, N1), shift.reshape(1, N1),
      w2, b2.reshape(1, N2))


def kernel(x, w1, b1, bn1_gamma, bn1_beta, bn1_rm, bn1_rv,
           w2, b2, bn2_gamma, bn2_beta, bn2_rm, bn2_rv,
           w3, b3, bn3_gamma, bn3_beta, bn3_rm, bn3_rv,
           fc1_w, fc1_b, bn4_gamma, bn4_beta, bn4_rm, bn4_rv,
           fc2_w, fc2_b):
    B, _, H1, W1 = x.shape
    H2, H3, H4 = H1 // 2, H1 // 6, H1 // 30
    Wp1 = -(-(W1 + 2) // 128) * 128

    xi = jnp.pad(x.astype(jnp.bfloat16),
                 ((0, 0), (0, 0), (1, 1), (1, Wp1 - W1 - 1)))
    xi = xi.reshape(B, x.shape[1], (H1 + 2) * Wp1)

    xi = _conv_stage(xi, w1, b1, bn1_gamma, bn1_beta, bn1_rm, bn1_rv,
                     k=2, Wp=Wp1, n=H2, pad_out=1)
    xi = _conv_stage(xi, w2, b2, bn2_gamma, bn2_beta, bn2_rm, bn2_rv,
                     k=3, Wp=128, n=H3, pad_out=1)
    o = _conv_stage(xi, w3, b3, bn3_gamma, bn3_beta, bn3_rm, bn3_rv,
                    k=5, Wp=128, n=H4, pad_out=0, out_dtype=jnp.float32)
    o = o.reshape(B, 32, H4, 128)[:, :, :, :H4]
    flat = o.reshape(B, -1)
    return _fc_head(flat, fc1_w, fc1_b, bn4_gamma, bn4_beta, bn4_rm, bn4_rv,
                    fc2_w, fc2_b)
